# baseline (speedup 1.0000x reference)
import functools

import jax
import jax.numpy as jnp
from jax import lax
from jax.experimental import pallas as pl
from jax.experimental.pallas import tpu as pltpu
from jax.experimental.pallas import tpu_sc as plsc

N_NODES = 10000
D_FEAT = 128
N_EDGES = 320000
E_TOT = 2 * N_EDGES

NC = 2
NS = 16
L = 16

F_PER_TILE = D_FEAT // NS
P_PER_TILE = F_PER_TILE // 2
E_PER_CORE = E_TOT // NC
CHUNK = 8000
N_CHUNKS = E_PER_CORE // CHUNK
GROUPS = CHUNK // L

_MESH = plsc.VectorSubcoreMesh(core_axis_name="c", subcore_axis_name="s")


@functools.partial(
    pl.kernel,
    out_type=jax.ShapeDtypeStruct((NS * E_TOT,), jnp.float32),
    mesh=_MESH,
    scratch_types=[
        pltpu.VMEM((P_PER_TILE * N_NODES,), jnp.int32),
        pltpu.VMEM((CHUNK,), jnp.int32),
        pltpu.VMEM((CHUNK,), jnp.int32),
        pltpu.VMEM((CHUNK,), jnp.float32),
    ],
    compiler_params=pltpu.CompilerParams(needs_layout_passes=False),
)
def _sc_partial_dots(ht_hbm, u_hbm, v_hbm, out_hbm, h_v, u_v, v_v, o_v):
    c = lax.axis_index("c")
    s = lax.axis_index("s")
    hbase = s * (P_PER_TILE * N_NODES)
    for p in range(5):
        pltpu.sync_copy(
            ht_hbm.at[pl.ds(hbase + p * 8000, 8000)], h_v.at[pl.ds(p * 8000, 8000)]
        )
    ebase = c * E_PER_CORE

    def chunk_body(k, carry):
        base = ebase + k * CHUNK
        pltpu.sync_copy(u_hbm.at[pl.ds(base, CHUNK)], u_v)
        pltpu.sync_copy(v_hbm.at[pl.ds(base, CHUNK)], v_v)

        @plsc.parallel_loop(0, CHUNK, step=L, unroll=4)
        def group_body(off):
            u = u_v[pl.ds(off, L)]
            w = v_v[pl.ds(off, L)]
            prods = []
            for q in range(P_PER_TILE):
                wu = plsc.load_gather(h_v, [u + (q * N_NODES)])
                ww = plsc.load_gather(h_v, [w + (q * N_NODES)])
                pu = plsc.bitcast(wu, jnp.bfloat16) * plsc.bitcast(ww, jnp.bfloat16)
                a, b = plsc.unpack(pu, format=plsc.PackFormat.INTERLEAVED)
                prods.append(a + b)
            while len(prods) > 1:
                prods = [prods[i] + prods[i + 1] for i in range(0, len(prods), 2)]
            o_v[pl.ds(off, L)] = prods[0]
        pltpu.sync_copy(o_v, out_hbm.at[pl.ds(s * E_TOT + base, CHUNK)])
        return carry

    lax.fori_loop(0, N_CHUNKS, chunk_body, 0)


ROWS_TOT = E_TOT // 128
POS_ROWS = N_EDGES // 128
BLK_ROWS = 200
NBLK = ROWS_TOT // BLK_ROWS


def _tc_loss_body(p_ref, acc_ref):
    i = pl.program_id(0)
    score = jnp.sum(p_ref[...], axis=0)
    row = lax.broadcasted_iota(jnp.int32, score.shape, 0) + i * BLK_ROWS
    t = jnp.where(row < POS_ROWS, -score, score)
    term = jnp.maximum(t, 0.0) + jnp.log1p(jnp.exp(-jnp.abs(t)))
    prev = jnp.where(i == 0, 0.0, acc_ref[0, 0])
    total = prev + jnp.sum(term)
    acc_ref[0, 0] = jnp.where(i == NBLK - 1, total / E_TOT, total)


_tc_loss = pl.pallas_call(
    _tc_loss_body,
    grid=(NBLK,),
    in_specs=[pl.BlockSpec((NS, BLK_ROWS, 128), lambda i: (0, i, 0))],
    out_specs=pl.BlockSpec(memory_space=pltpu.SMEM),
    out_shape=jax.ShapeDtypeStruct((1, 1), jnp.float32),
)


def kernel(block_outputs, pos_edge_index, neg_edge_index):
    ht_bf = block_outputs.astype(jnp.bfloat16).T
    pairs = ht_bf.reshape(D_FEAT // 2, 2, N_NODES).transpose(0, 2, 1)
    ht = lax.bitcast_convert_type(pairs, jnp.int32).reshape(-1)
    u = jnp.concatenate([pos_edge_index[0], neg_edge_index[0]])
    v = jnp.concatenate([pos_edge_index[1], neg_edge_index[1]])
    partials = _sc_partial_dots(ht, u, v).reshape(NS, ROWS_TOT, 128)
    return _tc_loss(partials)[0, 0]


# speedup vs baseline: 1.3796x; 1.3796x over previous
import functools

import jax
import jax.numpy as jnp
from jax import lax
from jax.experimental import pallas as pl
from jax.experimental.pallas import tpu as pltpu
from jax.experimental.pallas import tpu_sc as plsc

N_NODES = 10000
D_FEAT = 128
N_EDGES = 320000
E_TOT = 2 * N_EDGES

NC = 2
NS = 16
L = 16

F_PER_TILE = D_FEAT // NS
P_PER_TILE = F_PER_TILE // 2
E_PER_CORE = E_TOT // NC
CHUNK = 10000
N_CHUNKS = E_PER_CORE // CHUNK
N_PAIRS = N_CHUNKS // 2

_MESH = plsc.VectorSubcoreMesh(core_axis_name="c", subcore_axis_name="s")


@functools.partial(
    pl.kernel,
    out_type=jax.ShapeDtypeStruct((NS * E_TOT,), jnp.float32),
    mesh=_MESH,
    scratch_types=[
        pltpu.VMEM((P_PER_TILE * N_NODES,), jnp.int32),
        pltpu.VMEM((CHUNK,), jnp.int32),
        pltpu.VMEM((CHUNK,), jnp.int32),
        pltpu.VMEM((CHUNK,), jnp.float32),
        pltpu.VMEM((CHUNK,), jnp.int32),
        pltpu.VMEM((CHUNK,), jnp.int32),
        pltpu.VMEM((CHUNK,), jnp.float32),
        pltpu.SemaphoreType.DMA,
        pltpu.SemaphoreType.DMA,
        pltpu.SemaphoreType.DMA,
        pltpu.SemaphoreType.DMA,
    ],
    compiler_params=pltpu.CompilerParams(needs_layout_passes=False),
)
def _sc_partial_dots(
    ht_hbm, u_hbm, v_hbm, out_hbm,
    h_v, u0, v0, o0, u1, v1, o1, si0, si1, so0, so1,
):
    c = lax.axis_index("c")
    s = lax.axis_index("s")
    hbase = s * (P_PER_TILE * N_NODES)
    for p in range(5):
        pltpu.sync_copy(
            ht_hbm.at[pl.ds(hbase + p * 8000, 8000)], h_v.at[pl.ds(p * 8000, 8000)]
        )
    ebase = c * E_PER_CORE
    bufs = ((u0, v0, o0, si0, so0), (u1, v1, o1, si1, so1))

    def start_in(b, k):
        u_v, v_v, _, si, _ = bufs[b]
        base = ebase + k * CHUNK
        pltpu.async_copy(u_hbm.at[pl.ds(base, CHUNK)], u_v, si)
        pltpu.async_copy(v_hbm.at[pl.ds(base, CHUNK)], v_v, si)

    def wait_in(b):
        u_v, v_v, _, si, _ = bufs[b]
        pltpu.make_async_copy(u_hbm.at[pl.ds(0, CHUNK)], u_v, si).wait()
        pltpu.make_async_copy(v_hbm.at[pl.ds(0, CHUNK)], v_v, si).wait()

    def start_out(b, k):
        _, _, o_v, _, so = bufs[b]
        base = ebase + k * CHUNK
        pltpu.async_copy(o_v, out_hbm.at[pl.ds(s * E_TOT + base, CHUNK)], so)

    def wait_out(b):
        _, _, o_v, _, so = bufs[b]
        pltpu.make_async_copy(o_v, out_hbm.at[pl.ds(0, CHUNK)], so).wait()

    def compute(b):
        u_v, v_v, o_v, _, _ = bufs[b]

        @plsc.parallel_loop(0, CHUNK, step=L, unroll=4)
        def group_body(off):
            u = u_v[pl.ds(off, L)]
            w = v_v[pl.ds(off, L)]
            prods = []
            for q in range(P_PER_TILE):
                wu = plsc.load_gather(h_v, [u + (q * N_NODES)])
                ww = plsc.load_gather(h_v, [w + (q * N_NODES)])
                pu = plsc.bitcast(wu, jnp.bfloat16) * plsc.bitcast(ww, jnp.bfloat16)
                a, b = plsc.unpack(pu, format=plsc.PackFormat.INTERLEAVED)
                prods.append(a + b)
            while len(prods) > 1:
                prods = [prods[i] + prods[i + 1] for i in range(0, len(prods), 2)]
            o_v[pl.ds(off, L)] = prods[0]

    start_in(0, 0)

    def pair_body(j, carry):
        k0 = 2 * j
        start_in(1, k0 + 1)
        wait_in(0)

        @pl.when(j > 0)
        def _():
            wait_out(0)

        compute(0)
        start_out(0, k0)

        @pl.when(j < N_PAIRS - 1)
        def _():
            start_in(0, k0 + 2)

        wait_in(1)

        @pl.when(j > 0)
        def _():
            wait_out(1)

        compute(1)
        start_out(1, k0 + 1)
        return carry

    lax.fori_loop(0, N_PAIRS, pair_body, 0)
    wait_out(0)
    wait_out(1)


ROWS_TOT = E_TOT // 128
POS_ROWS = N_EDGES // 128
BLK_ROWS = 200
NBLK = ROWS_TOT // BLK_ROWS


def _tc_loss_body(p_ref, acc_ref):
    i = pl.program_id(0)
    score = jnp.sum(p_ref[...], axis=0)
    row = lax.broadcasted_iota(jnp.int32, score.shape, 0) + i * BLK_ROWS
    t = jnp.where(row < POS_ROWS, -score, score)
    term = jnp.maximum(t, 0.0) + jnp.log1p(jnp.exp(-jnp.abs(t)))
    prev = jnp.where(i == 0, 0.0, acc_ref[0, 0])
    total = prev + jnp.sum(term)
    acc_ref[0, 0] = jnp.where(i == NBLK - 1, total / E_TOT, total)


_tc_loss = pl.pallas_call(
    _tc_loss_body,
    grid=(NBLK,),
    in_specs=[pl.BlockSpec((NS, BLK_ROWS, 128), lambda i: (0, i, 0))],
    out_specs=pl.BlockSpec(memory_space=pltpu.SMEM),
    out_shape=jax.ShapeDtypeStruct((1, 1), jnp.float32),
)


def kernel(block_outputs, pos_edge_index, neg_edge_index):
    ht_bf = block_outputs.astype(jnp.bfloat16).T
    pairs = ht_bf.reshape(D_FEAT // 2, 2, N_NODES).transpose(0, 2, 1)
    ht = lax.bitcast_convert_type(pairs, jnp.int32).reshape(-1)
    u = jnp.concatenate([pos_edge_index[0], neg_edge_index[0]])
    v = jnp.concatenate([pos_edge_index[1], neg_edge_index[1]])
    partials = _sc_partial_dots(ht, u, v).reshape(NS, ROWS_TOT, 128)
    return _tc_loss(partials)[0, 0]


# speedup vs baseline: 1.5089x; 1.0937x over previous
import functools

import jax
import jax.numpy as jnp
from jax import lax
from jax.experimental import pallas as pl
from jax.experimental.pallas import tpu as pltpu
from jax.experimental.pallas import tpu_sc as plsc

N_NODES = 10000
D_FEAT = 128
N_EDGES = 320000
E_TOT = 2 * N_EDGES

NC = 2
NS = 16
L = 16

F_PER_TILE = D_FEAT // NS
P_PER_TILE = F_PER_TILE // 2
E_PER_CORE = E_TOT // NC
CHUNK = 10000
N_CHUNKS = E_PER_CORE // CHUNK
N_PAIRS = N_CHUNKS // 2

_MESH = plsc.VectorSubcoreMesh(core_axis_name="c", subcore_axis_name="s")


@functools.partial(
    pl.kernel,
    out_type=jax.ShapeDtypeStruct((NS * E_TOT,), jnp.float32),
    mesh=_MESH,
    scratch_types=[
        pltpu.VMEM((P_PER_TILE * N_NODES,), jnp.int32),
        pltpu.VMEM((CHUNK,), jnp.int32),
        pltpu.VMEM((CHUNK,), jnp.int32),
        pltpu.VMEM((CHUNK,), jnp.float32),
        pltpu.VMEM((CHUNK,), jnp.int32),
        pltpu.VMEM((CHUNK,), jnp.int32),
        pltpu.VMEM((CHUNK,), jnp.float32),
        pltpu.SemaphoreType.DMA,
        pltpu.SemaphoreType.DMA,
        pltpu.SemaphoreType.DMA,
        pltpu.SemaphoreType.DMA,
    ],
    compiler_params=pltpu.CompilerParams(needs_layout_passes=False),
)
def _sc_partial_dots(
    ht_hbm, pe_hbm, ne_hbm, out_hbm,
    h_v, u0, v0, o0, u1, v1, o1, si0, si1, so0, so1,
):
    c = lax.axis_index("c")
    s = lax.axis_index("s")
    hbase = s * (P_PER_TILE * N_NODES)
    for p in range(5):
        pltpu.sync_copy(
            ht_hbm.at[pl.ds(hbase + p * 8000, 8000)], h_v.at[pl.ds(p * 8000, 8000)]
        )
    ebase = c * E_PER_CORE
    bufs = ((u0, v0, o0, si0, so0), (u1, v1, o1, si1, so1))

    def start_in(b, k):
        u_v, v_v, _, si, _ = bufs[b]
        base = k * CHUNK

        @pl.when(c == 0)
        def _():
            pltpu.async_copy(pe_hbm.at[pl.ds(base, CHUNK)], u_v, si)
            pltpu.async_copy(pe_hbm.at[pl.ds(N_EDGES + base, CHUNK)], v_v, si)

        @pl.when(c == 1)
        def _():
            pltpu.async_copy(ne_hbm.at[pl.ds(base, CHUNK)], u_v, si)
            pltpu.async_copy(ne_hbm.at[pl.ds(N_EDGES + base, CHUNK)], v_v, si)

    def wait_in(b):
        u_v, v_v, _, si, _ = bufs[b]
        pltpu.make_async_copy(pe_hbm.at[pl.ds(0, CHUNK)], u_v, si).wait()
        pltpu.make_async_copy(pe_hbm.at[pl.ds(0, CHUNK)], v_v, si).wait()

    def start_out(b, k):
        _, _, o_v, _, so = bufs[b]
        base = ebase + k * CHUNK
        pltpu.async_copy(o_v, out_hbm.at[pl.ds(s * E_TOT + base, CHUNK)], so)

    def wait_out(b):
        _, _, o_v, _, so = bufs[b]
        pltpu.make_async_copy(o_v, out_hbm.at[pl.ds(0, CHUNK)], so).wait()

    def compute(b):
        u_v, v_v, o_v, _, _ = bufs[b]

        @plsc.parallel_loop(0, CHUNK, step=L, unroll=4)
        def group_body(off):
            u = u_v[pl.ds(off, L)]
            w = v_v[pl.ds(off, L)]
            prods = []
            for q in range(P_PER_TILE):
                wu = plsc.load_gather(h_v, [u + (q * N_NODES)])
                ww = plsc.load_gather(h_v, [w + (q * N_NODES)])
                pu = plsc.bitcast(wu, jnp.bfloat16) * plsc.bitcast(ww, jnp.bfloat16)
                a, b = plsc.unpack(pu, format=plsc.PackFormat.INTERLEAVED)
                prods.append(a + b)
            while len(prods) > 1:
                prods = [prods[i] + prods[i + 1] for i in range(0, len(prods), 2)]
            o_v[pl.ds(off, L)] = prods[0]

    start_in(0, 0)

    def pair_body(j, carry):
        k0 = 2 * j
        start_in(1, k0 + 1)
        wait_in(0)

        @pl.when(j > 0)
        def _():
            wait_out(0)

        compute(0)
        start_out(0, k0)

        @pl.when(j < N_PAIRS - 1)
        def _():
            start_in(0, k0 + 2)

        wait_in(1)

        @pl.when(j > 0)
        def _():
            wait_out(1)

        compute(1)
        start_out(1, k0 + 1)
        return carry

    lax.fori_loop(0, N_PAIRS, pair_body, 0)
    wait_out(0)
    wait_out(1)


ROWS_TOT = E_TOT // 128
POS_ROWS = N_EDGES // 128
BLK_ROWS = 200
NBLK = ROWS_TOT // BLK_ROWS


def _tc_loss_body(p_ref, acc_ref):
    i = pl.program_id(0)
    score = jnp.sum(p_ref[...], axis=0)
    row = lax.broadcasted_iota(jnp.int32, score.shape, 0) + i * BLK_ROWS
    t = jnp.where(row < POS_ROWS, -score, score)
    term = jnp.maximum(t, 0.0) + jnp.log1p(jnp.exp(-jnp.abs(t)))
    prev = jnp.where(i == 0, 0.0, acc_ref[0, 0])
    total = prev + jnp.sum(term)
    acc_ref[0, 0] = jnp.where(i == NBLK - 1, total / E_TOT, total)


_tc_loss = pl.pallas_call(
    _tc_loss_body,
    grid=(NBLK,),
    in_specs=[pl.BlockSpec((NS, BLK_ROWS, 128), lambda i: (0, i, 0))],
    out_specs=pl.BlockSpec(memory_space=pltpu.SMEM),
    out_shape=jax.ShapeDtypeStruct((1, 1), jnp.float32),
)


def kernel(block_outputs, pos_edge_index, neg_edge_index):
    packed = lax.bitcast_convert_type(
        block_outputs.astype(jnp.bfloat16).reshape(N_NODES, D_FEAT // 2, 2),
        jnp.int32,
    )
    ht = packed.T.reshape(-1)
    pe = pos_edge_index.reshape(-1)
    ne = neg_edge_index.reshape(-1)
    partials = _sc_partial_dots(ht, pe, ne).reshape(NS, ROWS_TOT, 128)
    return _tc_loss(partials)[0, 0]
